# baseline (speedup 1.0000x reference)
import math

import jax
import jax.numpy as jnp
from jax.experimental import pallas as pl
from jax.experimental.pallas import tpu as pltpu

D_MODEL = 2048
HIDDEN = 256
NUM_EXPERTS = 64
TOP_K = 2
N_TOK = 16384

CH = 1024
NBUF = 4
N_CHUNK = N_TOK // CH

_INV_SQRT2 = 1.0 / math.sqrt(2.0)


def _gating(logits):
    col = jax.lax.broadcasted_iota(jnp.int32, logits.shape, 1)
    revcol = (NUM_EXPERTS - 1 - col).astype(jnp.float32)
    m1 = jnp.max(logits, axis=1, keepdims=True)
    r1 = jnp.max(jnp.where(logits == m1, revcol, -1.0), axis=1,
                 keepdims=True)
    i1 = (NUM_EXPERTS - 1) - r1.astype(jnp.int32)
    masked = jnp.where(col == i1, -jnp.inf, logits)
    m2 = jnp.max(masked, axis=1, keepdims=True)
    r2 = jnp.max(jnp.where(masked == m2, revcol, -1.0), axis=1,
                 keepdims=True)
    i2 = (NUM_EXPERTS - 1) - r2.astype(jnp.int32)
    e2 = jnp.exp(m2 - m1)
    denom = 1.0 + e2
    weights = jnp.concatenate([1.0 / denom, e2 / denom], axis=1)
    idx = jnp.concatenate([i1, i2], axis=1)
    return weights, idx


def _router_body(x_hbm, w1_ref, b1_ref, w2_ref, b2_ref,
                 w_out_ref, i_out_ref, buf, sems):
    def issue(chunk, slot):
        pltpu.make_async_copy(
            x_hbm.at[pl.ds(chunk * CH, CH), :], buf.at[slot],
            sems.at[slot]).start()

    for s in range(NBUF):
        issue(s, s)

    for i in range(N_CHUNK):
        slot = i % NBUF
        pltpu.make_async_copy(
            x_hbm.at[pl.ds(i * CH, CH), :], buf.at[slot],
            sems.at[slot]).wait()

        h = jnp.dot(buf[slot], w1_ref[...],
                    preferred_element_type=jnp.float32)
        h = h + b1_ref[...]
        h = 0.5 * h * (1.0 + jax.lax.erf(h * _INV_SQRT2))
        logits = jnp.dot(h, w2_ref[...],
                         preferred_element_type=jnp.float32)
        logits = logits + b2_ref[...]
        weights, idx = _gating(logits)
        w_out_ref[pl.ds(i * CH, CH), :] = weights
        i_out_ref[pl.ds(i * CH, CH), :] = idx

        if i + NBUF < N_CHUNK:
            issue(i + NBUF, slot)


@jax.jit
def kernel(x, W1, b1, W2, b2):
    b1r = b1.reshape(1, HIDDEN)
    b2r = b2.reshape(1, NUM_EXPERTS)
    weights, topk_i = pl.pallas_call(
        _router_body,
        in_specs=[
            pl.BlockSpec(memory_space=pl.ANY),
            pl.BlockSpec(memory_space=pltpu.MemorySpace.VMEM),
            pl.BlockSpec(memory_space=pltpu.MemorySpace.VMEM),
            pl.BlockSpec(memory_space=pltpu.MemorySpace.VMEM),
            pl.BlockSpec(memory_space=pltpu.MemorySpace.VMEM),
        ],
        out_specs=[
            pl.BlockSpec(memory_space=pltpu.MemorySpace.VMEM),
            pl.BlockSpec(memory_space=pltpu.MemorySpace.VMEM),
        ],
        out_shape=[
            jax.ShapeDtypeStruct((N_TOK, TOP_K), jnp.float32),
            jax.ShapeDtypeStruct((N_TOK, TOP_K), jnp.int32),
        ],
        scratch_shapes=[
            pltpu.VMEM((NBUF, CH, D_MODEL), jnp.float32),
            pltpu.SemaphoreType.DMA((NBUF,)),
        ],
        compiler_params=pltpu.CompilerParams(
            vmem_limit_bytes=100 * 1024 * 1024,
        ),
    )(x, W1, b1r, W2, b2r)
    return (weights, topk_i)


# speedup vs baseline: 1.1296x; 1.1296x over previous
import math

import jax
import jax.numpy as jnp
from jax.experimental import pallas as pl
from jax.experimental.pallas import tpu as pltpu

D_MODEL = 2048
HIDDEN = 256
NUM_EXPERTS = 64
TOP_K = 2
N_TOK = 16384

TILE = 2048
HALF = D_MODEL // 2

_INV_SQRT2 = 1.0 / math.sqrt(2.0)


def _fused_gating_kernel(xa_ref, xb_ref, w1a_ref, w1b_ref, b1_ref,
                         w2_ref, b2_ref, w_out_ref, i_out_ref):
    h = jnp.dot(xa_ref[...], w1a_ref[...],
                preferred_element_type=jnp.float32)
    h = h + jnp.dot(xb_ref[...], w1b_ref[...],
                    preferred_element_type=jnp.float32)
    h = h + b1_ref[...]
    h = 0.5 * h * (1.0 + jax.lax.erf(h * _INV_SQRT2))
    logits = jnp.dot(h, w2_ref[...], preferred_element_type=jnp.float32)
    logits = logits + b2_ref[...]

    col = jax.lax.broadcasted_iota(jnp.int32, logits.shape, 1)
    revcol = (NUM_EXPERTS - 1 - col).astype(jnp.float32)
    m1 = jnp.max(logits, axis=1, keepdims=True)
    r1 = jnp.max(jnp.where(logits == m1, revcol, -1.0), axis=1,
                 keepdims=True)
    i1 = (NUM_EXPERTS - 1) - r1.astype(jnp.int32)
    masked = jnp.where(col == i1, -jnp.inf, logits)
    m2 = jnp.max(masked, axis=1, keepdims=True)
    r2 = jnp.max(jnp.where(masked == m2, revcol, -1.0), axis=1,
                 keepdims=True)
    i2 = (NUM_EXPERTS - 1) - r2.astype(jnp.int32)

    e2 = jnp.exp(m2 - m1)
    denom = 1.0 + e2
    w_out_ref[...] = jnp.concatenate([1.0 / denom, e2 / denom], axis=1)
    i_out_ref[...] = jnp.concatenate([i1, i2], axis=1)


@jax.jit
def kernel(x, W1, b1, W2, b2):
    b1r = b1.reshape(1, HIDDEN)
    b2r = b2.reshape(1, NUM_EXPERTS)
    grid = (N_TOK // TILE,)
    weights, topk_i = pl.pallas_call(
        _fused_gating_kernel,
        grid=grid,
        in_specs=[
            pl.BlockSpec((TILE, HALF), lambda i: (i, 0)),
            pl.BlockSpec((TILE, HALF), lambda i: (i, 1)),
            pl.BlockSpec((HALF, HIDDEN), lambda i: (0, 0)),
            pl.BlockSpec((HALF, HIDDEN), lambda i: (1, 0)),
            pl.BlockSpec((1, HIDDEN), lambda i: (0, 0)),
            pl.BlockSpec((HIDDEN, NUM_EXPERTS), lambda i: (0, 0)),
            pl.BlockSpec((1, NUM_EXPERTS), lambda i: (0, 0)),
        ],
        out_specs=[
            pl.BlockSpec((TILE, TOP_K), lambda i: (i, 0)),
            pl.BlockSpec((TILE, TOP_K), lambda i: (i, 0)),
        ],
        out_shape=[
            jax.ShapeDtypeStruct((N_TOK, TOP_K), jnp.float32),
            jax.ShapeDtypeStruct((N_TOK, TOP_K), jnp.int32),
        ],
        compiler_params=pltpu.CompilerParams(
            dimension_semantics=("arbitrary",),
        ),
    )(x, x, W1, W1, b1r, W2, b2r)
    return (weights, topk_i)
